# Initial kernel scaffold; baseline (speedup 1.0000x reference)
import jax
import jax.numpy as jnp
from jax.experimental import pallas as pl


def kernel(input_ids, attention_mask, table, W, b):
    raise NotImplementedError("write your pallas kernel here")


# speedup vs baseline: 99.8834x; 99.8834x over previous
import functools

import jax
import jax.numpy as jnp
from jax import lax
from jax.experimental import pallas as pl
from jax.experimental.pallas import tpu as pltpu
from jax.experimental.pallas import tpu_sc as plsc

B = 16384
L_SEQ = 200
VOCAB = 20
EMB = 4
NUM_OUT = 2

NUM_CORES = 2
NUM_SUBCORES = 16
LANES = 16
NUM_TILES = NUM_CORES * NUM_SUBCORES
ROWS_PER_TILE = B // NUM_TILES
GROUPS = ROWS_PER_TILE // LANES


def _tc_body(ids_hbm, tab_hbm, w_hbm, b_hbm, out_hbm, ids_v, tab_v, w_v,
             b_v, counts, out_v):
  wid = lax.axis_index("s") * NUM_CORES + lax.axis_index("c")
  base = wid * ROWS_PER_TILE

  pltpu.sync_copy(ids_hbm.at[pl.ds(base, ROWS_PER_TILE)], ids_v)
  pltpu.sync_copy(tab_hbm, tab_v)
  pltpu.sync_copy(w_hbm, w_v)
  pltpu.sync_copy(b_hbm, b_v)

  wvec = w_v[:]
  w = [wvec[i] for i in range(NUM_OUT * EMB)]
  tvecs = [tab_v[pl.ds(j * LANES, LANES)] for j in range(VOCAB * EMB // LANES)]
  t_all = [tvecs[i // LANES][i % LANES] for i in range(VOCAB * EMB)]
  v0 = []
  v1 = []
  for k in range(VOCAB):
    t = t_all[k * EMB:(k + 1) * EMB]
    v0.append(t[0] * w[0] + t[1] * w[1] + t[2] * w[2] + t[3] * w[3])
    v1.append(t[0] * w[4] + t[1] * w[5] + t[2] * w[6] + t[3] * w[7])
  inv_l = 1.0 / L_SEQ
  bvec = b_v[:]
  b0 = bvec[0]
  b1 = bvec[1]

  lane = lax.iota(jnp.int32, LANES)
  ones_f = jnp.ones((LANES,), jnp.float32)
  zero_i = jnp.zeros((LANES,), jnp.int32)
  one_i = jnp.ones((LANES,), jnp.int32)

  @pl.loop(0, GROUPS)
  def _group(g):
    row16 = g * LANES + lane
    for k in range(VOCAB):
      counts[k] = jnp.zeros((LANES,), jnp.float32)

    @pl.loop(0, L_SEQ, unroll=8)
    def _tok(l):
      col = jnp.full((LANES,), l, jnp.int32)
      ids16 = plsc.load_gather(ids_v, [row16, col])
      plsc.addupdate_scatter(counts, [ids16, lane], ones_f)

    acc0 = jnp.zeros((LANES,), jnp.float32)
    acc1 = jnp.zeros((LANES,), jnp.float32)
    for k in range(VOCAB):
      cnt = counts[k]
      acc0 = acc0 + cnt * v0[k]
      acc1 = acc1 + cnt * v1[k]
    plsc.store_scatter(out_v, [row16, zero_i], acc0 * inv_l + b0)
    plsc.store_scatter(out_v, [row16, one_i], acc1 * inv_l + b1)

  pltpu.sync_copy(out_v, out_hbm.at[pl.ds(base, ROWS_PER_TILE)])


@jax.jit
def _run(ids, tab_flat, w_flat, b_flat):
  mesh = plsc.VectorSubcoreMesh(
      core_axis_name="c", subcore_axis_name="s",
      num_cores=NUM_CORES, num_subcores=NUM_SUBCORES)
  fn = pl.kernel(
      _tc_body,
      out_type=jax.ShapeDtypeStruct((B, NUM_OUT), jnp.float32),
      mesh=mesh,
      compiler_params=pltpu.CompilerParams(
          use_tc_tiling_on_sc=False, needs_layout_passes=False),
      scratch_types=[
          pltpu.VMEM((ROWS_PER_TILE, L_SEQ), jnp.int32),
          pltpu.VMEM((VOCAB * EMB,), jnp.float32),
          pltpu.VMEM((LANES,), jnp.float32),
          pltpu.VMEM((LANES,), jnp.float32),
          pltpu.VMEM((VOCAB, LANES), jnp.float32),
          pltpu.VMEM((ROWS_PER_TILE, NUM_OUT), jnp.float32),
      ],
  )
  return fn(ids, tab_flat, w_flat, b_flat)


def kernel(input_ids, attention_mask, table, W, b):
  del attention_mask
  ids = input_ids.astype(jnp.int32)
  tab_flat = table.reshape(-1).astype(jnp.float32)
  w_flat = jnp.pad(W.reshape(-1).astype(jnp.float32), (0, LANES - W.size))
  b_flat = jnp.pad(b.astype(jnp.float32), (0, LANES - b.size))
  return _run(ids, tab_flat, w_flat, b_flat)


# speedup vs baseline: 159.3395x; 1.5953x over previous
import functools

import jax
import jax.numpy as jnp
from jax import lax
from jax.experimental import pallas as pl
from jax.experimental.pallas import tpu as pltpu
from jax.experimental.pallas import tpu_sc as plsc

B = 16384
L_SEQ = 200
VOCAB = 20
EMB = 4
NUM_OUT = 2

NUM_CORES = 2
NUM_SUBCORES = 16
LANES = 16
NUM_TILES = NUM_CORES * NUM_SUBCORES
ROWS_PER_TILE = B // NUM_TILES
GROUPS = ROWS_PER_TILE // LANES


def _tc_body(ids_hbm, tab_hbm, w_hbm, b_hbm, out_hbm, ids_v, tab_v, w_v,
             b_v, counts, out_v):
  wid = lax.axis_index("s") * NUM_CORES + lax.axis_index("c")
  base = wid * ROWS_PER_TILE

  pltpu.sync_copy(ids_hbm.at[pl.ds(base, ROWS_PER_TILE)], ids_v)
  pltpu.sync_copy(tab_hbm, tab_v)
  pltpu.sync_copy(w_hbm, w_v)
  pltpu.sync_copy(b_hbm, b_v)

  wvec = w_v[:]
  w = [wvec[i] for i in range(NUM_OUT * EMB)]
  tvecs = [tab_v[pl.ds(j * LANES, LANES)] for j in range(VOCAB * EMB // LANES)]
  t_all = [tvecs[i // LANES][i % LANES] for i in range(VOCAB * EMB)]
  v0 = []
  v1 = []
  for k in range(VOCAB):
    t = t_all[k * EMB:(k + 1) * EMB]
    v0.append(t[0] * w[0] + t[1] * w[1] + t[2] * w[2] + t[3] * w[3])
    v1.append(t[0] * w[4] + t[1] * w[5] + t[2] * w[6] + t[3] * w[7])
  inv_l = 1.0 / L_SEQ
  bvec = b_v[:]
  b0 = bvec[0]
  b1 = bvec[1]

  lane = lax.iota(jnp.int32, LANES)
  ones_f = jnp.ones((LANES,), jnp.float32)
  zero_i = jnp.zeros((LANES,), jnp.int32)
  one_i = jnp.ones((LANES,), jnp.int32)

  @pl.loop(0, GROUPS)
  def _group(g):
    row16 = g * LANES + lane
    for k in range(VOCAB):
      counts[k] = jnp.zeros((LANES,), jnp.float32)

    @plsc.parallel_loop(0, L_SEQ, unroll=8)
    def _tok(l):
      col = jnp.full((LANES,), l, jnp.int32)
      ids16 = plsc.load_gather(ids_v, [row16, col])
      plsc.addupdate_scatter(counts, [ids16, lane], ones_f)

    acc0 = jnp.zeros((LANES,), jnp.float32)
    acc1 = jnp.zeros((LANES,), jnp.float32)
    for k in range(VOCAB):
      cnt = counts[k]
      acc0 = acc0 + cnt * v0[k]
      acc1 = acc1 + cnt * v1[k]
    plsc.store_scatter(out_v, [row16, zero_i], acc0 * inv_l + b0)
    plsc.store_scatter(out_v, [row16, one_i], acc1 * inv_l + b1)

  pltpu.sync_copy(out_v, out_hbm.at[pl.ds(base, ROWS_PER_TILE)])


@jax.jit
def _run(ids, tab_flat, w_flat, b_flat):
  mesh = plsc.VectorSubcoreMesh(
      core_axis_name="c", subcore_axis_name="s",
      num_cores=NUM_CORES, num_subcores=NUM_SUBCORES)
  fn = pl.kernel(
      _tc_body,
      out_type=jax.ShapeDtypeStruct((B, NUM_OUT), jnp.float32),
      mesh=mesh,
      compiler_params=pltpu.CompilerParams(
          use_tc_tiling_on_sc=False, needs_layout_passes=False),
      scratch_types=[
          pltpu.VMEM((ROWS_PER_TILE, L_SEQ), jnp.int32),
          pltpu.VMEM((VOCAB * EMB,), jnp.float32),
          pltpu.VMEM((LANES,), jnp.float32),
          pltpu.VMEM((LANES,), jnp.float32),
          pltpu.VMEM((VOCAB, LANES), jnp.float32),
          pltpu.VMEM((ROWS_PER_TILE, NUM_OUT), jnp.float32),
      ],
  )
  return fn(ids, tab_flat, w_flat, b_flat)


def kernel(input_ids, attention_mask, table, W, b):
  del attention_mask
  ids = input_ids.astype(jnp.int32)
  tab_flat = table.reshape(-1).astype(jnp.float32)
  w_flat = jnp.pad(W.reshape(-1).astype(jnp.float32), (0, LANES - W.size))
  b_flat = jnp.pad(b.astype(jnp.float32), (0, LANES - b.size))
  return _run(ids, tab_flat, w_flat, b_flat)
